# baseline (speedup 1.0000x reference)
import jax
import jax.numpy as jnp
from jax.experimental import pallas as pl

_BS, _NV, _NP, _FD = 64, 4, 512, 10
_THR = 6.0
_SCALE = 1.0


def _chamfer_body(goal_ref, stateT_ref, out_ref):
    acc = None
    for v in range(_NV):
        g = goal_ref[0, v]
        sA = stateT_ref[0, v]

        P = None
        for f in range(5, 9):
            d = g[:, f:f + 1] - sA[f:f + 1, :]
            P = d * d if P is None else P + d * d

        spk = jax.lax.bitcast_convert_type(sA[10:11, :], jnp.int32)
        gpk = jax.lax.bitcast_convert_type(g[:, 10:11], jnp.int32)

        minv_g = jnp.min(P, axis=1, keepdims=True)
        sel = P == minv_g
        q1 = jnp.sum(jnp.where(sel, spk, 0), axis=1, keepdims=True)
        q1r = jnp.reshape(q1, (1, _NP))
        m1r = jnp.reshape(minv_g, (1, _NP))
        sx = jax.lax.bitcast_convert_type(q1r & -65536, jnp.float32)
        sy = jax.lax.bitcast_convert_type(q1r << 16, jnp.float32)
        dx = sA[11:12, :] - sx
        dy = sA[12:13, :] - sy
        xy1 = jnp.where(m1r > _THR, 1.0, jnp.sqrt(dx * dx + dy * dy))

        minv_s = jnp.min(P, axis=0, keepdims=True)
        sel2 = P == minv_s
        q2 = jnp.sum(jnp.where(sel2, gpk, 0), axis=0, keepdims=True)
        gx = jax.lax.bitcast_convert_type(q2 & -65536, jnp.float32)
        gy = jax.lax.bitcast_convert_type(q2 << 16, jnp.float32)
        dx2 = sA[0:1, :] - gx
        dy2 = sA[1:2, :] - gy
        xy2 = jnp.where(minv_s > _THR, 1.0, jnp.sqrt(dx2 * dx2 + dy2 * dy2))

        part = xy1 + xy2
        acc = part if acc is None else acc + part

    total = jnp.sum(acc)
    out_ref[...] = (total * (-_SCALE / (2.0 * _NP * _NV))).reshape(1, 1, 1)


def _pack_xy(t):
    xb = t[..., 0].astype(jnp.bfloat16)
    yb = t[..., 1].astype(jnp.bfloat16)
    xu = jax.lax.bitcast_convert_type(xb, jnp.uint16).astype(jnp.uint32)
    yu = jax.lax.bitcast_convert_type(yb, jnp.uint16).astype(jnp.uint32)
    pk = (xu << 16) | yu
    return jax.lax.bitcast_convert_type(pk, jnp.float32)[..., None]


@jax.jit
def kernel(achieved_goal, desired_goal):
    goal = jnp.concatenate([desired_goal, _pack_xy(desired_goal)], axis=-1)
    state_aug = jnp.concatenate(
        [achieved_goal, _pack_xy(achieved_goal), desired_goal[..., 0:2]],
        axis=-1)
    stateT = jnp.swapaxes(state_aug, -1, -2)
    out = pl.pallas_call(
        _chamfer_body,
        grid=(_BS,),
        in_specs=[
            pl.BlockSpec((1, _NV, _NP, 11), lambda b: (b, 0, 0, 0)),
            pl.BlockSpec((1, _NV, 13, _NP), lambda b: (b, 0, 0, 0)),
        ],
        out_specs=pl.BlockSpec((1, 1, 1), lambda b: (b, 0, 0)),
        out_shape=jax.ShapeDtypeStruct((_BS, 1, 1), jnp.float32),
    )(goal, stateT)
    return out.reshape(_BS, 1)
